# baseline (speedup 1.0000x reference)
import functools

import jax
import jax.numpy as jnp
from jax import lax
from jax.experimental import pallas as pl
from jax.experimental.pallas import tpu as pltpu
from jax.experimental.pallas import tpu_sc as plsc

_M = 0.25
_GAMMA = 64.0
_EPS = 1e-6
_D = 131072
_NW = 32
_NCHUNK = 16
_CHUNK = _D // _NCHUNK


def _sc_partial_sums(anchor2d, negative2d, idx):
    mesh = plsc.VectorSubcoreMesh(core_axis_name="c", subcore_axis_name="s")

    @functools.partial(
        pl.kernel,
        out_type=jax.ShapeDtypeStruct((_NW, 16), jnp.float32),
        mesh=mesh,
        scratch_types=[
            pltpu.VMEM((1,), jnp.int32),
            pltpu.VMEM((1, _CHUNK), jnp.float32),
            pltpu.VMEM((1, _CHUNK), jnp.float32),
            pltpu.VMEM((16,), jnp.float32),
            pltpu.SemaphoreType.DMA,
            pltpu.SemaphoreType.DMA,
        ],
    )
    def body(a_hbm, n_hbm, idx_hbm, out_hbm, idx_v, a_v, n_v, acc_v,
             sem_a, sem_n):
        wid = lax.axis_index("s") * 2 + lax.axis_index("c")
        pltpu.sync_copy(idx_hbm.at[wid], idx_v)
        cp_a = pltpu.async_copy(a_hbm.at[idx_v], a_v, sem_a)
        cp_n = pltpu.async_copy(n_hbm.at[idx_v], n_v, sem_n)
        cp_a.wait()
        cp_n.wait()

        def step(i, acc):
            a = a_v[0, pl.ds(pl.multiple_of(i * 16, 16), 16)]
            n = n_v[0, pl.ds(pl.multiple_of(i * 16, 16), 16)]
            d = a - n + _EPS
            return acc + d * d

        acc = lax.fori_loop(0, _CHUNK // 16, step,
                            jnp.zeros((16,), jnp.float32))
        acc_v[...] = acc
        pltpu.sync_copy(acc_v, out_hbm.at[wid])

    return body(anchor2d, negative2d, idx)


def _tc_finish(partials2):
    def body(p_ref, o_ref):
        s = jnp.sum(p_ref[...], axis=1)
        logit = _GAMMA * (s - _M * _M)
        y = jnp.maximum(logit, 0.0)
        soft = y + jnp.log1p(jnp.exp(-y))
        o_ref[0, 0] = 0.5 * (soft[0] + soft[1])

    return pl.pallas_call(
        body,
        out_shape=jax.ShapeDtypeStruct((1, 1), jnp.float32),
        out_specs=pl.BlockSpec(memory_space=pltpu.SMEM),
    )(partials2)


def kernel(anchor, positive, negative, labels):
    del positive
    a2d = anchor.reshape(8 * _NCHUNK, _CHUNK)
    n2d = negative.reshape(8 * _NCHUNK, _CHUNK)
    e = (labels[0] != labels[1]).astype(jnp.int32)
    r0 = 3 * e
    r1 = 4 * e
    t = jnp.arange(_NCHUNK, dtype=jnp.int32)
    idx = jnp.concatenate([r0 * _NCHUNK + t, r1 * _NCHUNK + t])
    partials = _sc_partial_sums(a2d, n2d, idx.reshape(_NW, 1))
    loss = _tc_finish(partials.reshape(2, 16 * _NW // 2))
    return loss[0, 0]


# speedup vs baseline: 1.0272x; 1.0272x over previous
import functools

import jax
import jax.numpy as jnp
from jax import lax
from jax.experimental import pallas as pl
from jax.experimental.pallas import tpu as pltpu
from jax.experimental.pallas import tpu_sc as plsc

_M = 0.25
_GAMMA = 64.0
_EPS = 1e-6
_D = 131072
_NS = 16
_CHUNK = _D // _NS
_UNROLL = 8


def _sc_loss_terms(anchor2d, negative2d, labels2):
    mesh = plsc.VectorSubcoreMesh(core_axis_name="c", subcore_axis_name="s")

    @functools.partial(
        pl.kernel,
        out_type=(jax.ShapeDtypeStruct((2, 16), jnp.float32),
                  jax.ShapeDtypeStruct((2, _NS, 16), jnp.float32)),
        mesh=mesh,
        scratch_types=[
            pltpu.VMEM((16,), jnp.int32),
            pltpu.VMEM((_CHUNK,), jnp.float32),
            pltpu.VMEM((_CHUNK,), jnp.float32),
            pltpu.VMEM((16,), jnp.float32),
            pltpu.VMEM((16, 16), jnp.float32),
            pltpu.SemaphoreType.DMA,
            pltpu.SemaphoreType.DMA,
        ],
    )
    def body(a_hbm, n_hbm, lab_hbm, out_hbm, part_hbm, lab_v, a_v, n_v,
             acc_v, part_v, sem_a, sem_n):
        cid = lax.axis_index("c")
        sid = lax.axis_index("s")
        pltpu.sync_copy(lab_hbm, lab_v)
        lv = lab_v[...]
        e = (lv[0] != lv[1]).astype(jnp.int32)
        row = ((3 + cid) * e) * _NS + sid
        pltpu.sync_copy(a_hbm.at[row], a_v)
        pltpu.sync_copy(n_hbm.at[row], n_v)

        def step(i, accs):
            base = pl.multiple_of(i * (16 * _UNROLL), 16 * _UNROLL)
            new = []
            for j in range(_UNROLL):
                a = a_v[pl.ds(base + j * 16, 16)]
                n = n_v[pl.ds(base + j * 16, 16)]
                d = a - n + _EPS
                new.append(accs[j] + d * d)
            return tuple(new)

        zeros = jnp.zeros((16,), jnp.float32)
        accs = lax.fori_loop(0, _CHUNK // (16 * _UNROLL), step,
                             (zeros,) * _UNROLL)
        acc = accs[0]
        for j in range(1, _UNROLL):
            acc = acc + accs[j]
        acc_v[...] = acc
        pltpu.sync_copy(acc_v, part_hbm.at[cid, sid])
        plsc.subcore_barrier()

        @pl.when(sid == 0)
        def _():
            pltpu.sync_copy(part_hbm.at[cid], part_v)
            tot = part_v[0]
            for i in range(1, _NS):
                tot = tot + part_v[i]
            lane = lax.iota(jnp.int32, 16)
            for sh in (8, 4, 2, 1):
                tot = tot + tot.at[lane ^ sh].get(mode="promise_in_bounds")
            y16 = jnp.maximum(_GAMMA * (tot - _M * _M), 0.0)
            t = jnp.exp(-y16)
            z = t * (1.0 - t * (0.5 - t * (1.0 / 3.0 - t * 0.25)))
            for _ in range(3):
                z = z - 1.0 + (1.0 + t) * jnp.exp(-z)
            acc_v[...] = y16 + z
            pltpu.sync_copy(acc_v, out_hbm.at[cid])

    return body(anchor2d, negative2d, labels2)[0]


def kernel(anchor, positive, negative, labels):
    del positive
    a2d = anchor.reshape(8 * _NS, _CHUNK)
    n2d = negative.reshape(8 * _NS, _CHUNK)
    lab16 = jnp.zeros((16,), jnp.int32).at[:2].set(labels.astype(jnp.int32))
    terms = _sc_loss_terms(a2d, n2d, lab16)
    return 0.5 * (terms[0, 0] + terms[1, 0])


# speedup vs baseline: 1.1060x; 1.0768x over previous
import functools

import jax
import jax.numpy as jnp
from jax import lax
from jax.experimental import pallas as pl
from jax.experimental.pallas import tpu as pltpu
from jax.experimental.pallas import tpu_sc as plsc

_M = 0.25
_GAMMA = 64.0
_EPS = 1e-6
_D = 131072
_NS = 16
_CHUNK = _D // _NS
_UNROLL = 8


def _sc_loss_terms(anchor2d, negative2d, labels2):
    mesh = plsc.VectorSubcoreMesh(core_axis_name="c", subcore_axis_name="s")

    @functools.partial(
        pl.kernel,
        out_type=(jax.ShapeDtypeStruct((2, 16), jnp.float32),
                  jax.ShapeDtypeStruct((2, _NS, 16), jnp.float32)),
        mesh=mesh,
        scratch_types=[
            pltpu.VMEM((16,), jnp.int32),
            pltpu.VMEM((_CHUNK,), jnp.float32),
            pltpu.VMEM((_CHUNK,), jnp.float32),
            pltpu.VMEM((16,), jnp.float32),
            pltpu.VMEM((16, 16), jnp.float32),
            pltpu.SemaphoreType.DMA,
            pltpu.SemaphoreType.DMA,
            pltpu.SemaphoreType.DMA,
            pltpu.SemaphoreType.DMA,
        ],
    )
    def body(a_hbm, n_hbm, lab_hbm, out_hbm, part_hbm, lab_v, a_v, n_v,
             acc_v, part_v, sem_a, sem_n, sem_a1, sem_n1):
        cid = lax.axis_index("c")
        sid = lax.axis_index("s")
        pltpu.sync_copy(lab_hbm, lab_v.at[pl.ds(0, 2)])
        lv = lab_v[...]
        e = (lv[0] != lv[1]).astype(jnp.int32)
        row = ((3 + cid) * e) * _NS + sid
        half = _CHUNK // 2
        cp_a0 = pltpu.async_copy(
            a_hbm.at[row, pl.ds(0, half)], a_v.at[pl.ds(0, half)], sem_a)
        cp_n0 = pltpu.async_copy(
            n_hbm.at[row, pl.ds(0, half)], n_v.at[pl.ds(0, half)], sem_n)
        cp_a1 = pltpu.async_copy(
            a_hbm.at[row, pl.ds(half, half)], a_v.at[pl.ds(half, half)],
            sem_a1)
        cp_n1 = pltpu.async_copy(
            n_hbm.at[row, pl.ds(half, half)], n_v.at[pl.ds(half, half)],
            sem_n1)

        def step(i, accs):
            base = pl.multiple_of(i * (16 * _UNROLL), 16 * _UNROLL)
            new = []
            for j in range(_UNROLL):
                a = a_v[pl.ds(base + j * 16, 16)]
                n = n_v[pl.ds(base + j * 16, 16)]
                d = a - n + _EPS
                new.append(accs[j] + d * d)
            return tuple(new)

        zeros = jnp.zeros((16,), jnp.float32)
        nsteps = half // (16 * _UNROLL)
        cp_a0.wait()
        cp_n0.wait()
        accs = lax.fori_loop(0, nsteps, step, (zeros,) * _UNROLL)
        cp_a1.wait()
        cp_n1.wait()
        accs = lax.fori_loop(nsteps, 2 * nsteps, step, accs)
        acc = accs[0]
        for j in range(1, _UNROLL):
            acc = acc + accs[j]
        acc_v[...] = acc
        pltpu.sync_copy(acc_v, part_hbm.at[cid, sid])
        plsc.subcore_barrier()

        @pl.when(sid == 0)
        def _():
            pltpu.sync_copy(part_hbm.at[cid], part_v)
            tot = part_v[0]
            for i in range(1, _NS):
                tot = tot + part_v[i]
            lane = lax.iota(jnp.int32, 16)
            for sh in (8, 4, 2, 1):
                tot = tot + tot.at[lane ^ sh].get(mode="promise_in_bounds")
            y16 = jnp.maximum(_GAMMA * (tot - _M * _M), 0.0)
            t = jnp.exp(-y16)
            z = t * (1.0 - t * (0.5 - t * (1.0 / 3.0 - t * 0.25)))
            for _ in range(3):
                z = z - 1.0 + (1.0 + t) * jnp.exp(-z)
            acc_v[...] = y16 + z
            pltpu.sync_copy(acc_v, out_hbm.at[cid])

    return body(anchor2d, negative2d, labels2)[0]


def kernel(anchor, positive, negative, labels):
    del positive
    a2d = anchor.reshape(8 * _NS, _CHUNK)
    n2d = negative.reshape(8 * _NS, _CHUNK)
    terms = _sc_loss_terms(a2d, n2d, labels.astype(jnp.int32))
    return 0.5 * (terms[0, 0] + terms[1, 0])


# speedup vs baseline: 1.1263x; 1.0183x over previous
import functools

import jax
import jax.numpy as jnp
from jax import lax
from jax.experimental import pallas as pl
from jax.experimental.pallas import tpu as pltpu
from jax.experimental.pallas import tpu_sc as plsc

_M = 0.25
_GAMMA = 64.0
_EPS = 1e-6
_D = 131072
_NS = 16
_CHUNK = _D // _NS
_UNROLL = 8


def _sc_loss_terms(anchor2d, negative2d, labels2):
    mesh = plsc.VectorSubcoreMesh(core_axis_name="c", subcore_axis_name="s")

    @functools.partial(
        pl.kernel,
        out_type=jax.ShapeDtypeStruct((2, 16), jnp.float32),
        mesh=mesh,
        scratch_types=[
            pltpu.VMEM((16,), jnp.int32),
            pltpu.VMEM((_CHUNK,), jnp.float32),
            pltpu.VMEM((_CHUNK,), jnp.float32),
            pltpu.VMEM((16,), jnp.float32),
            pltpu.VMEM((16,), jnp.int32),
            pltpu.VMEM_SHARED((16,), jnp.float32),
            pltpu.SemaphoreType.DMA,
            pltpu.SemaphoreType.DMA,
            pltpu.SemaphoreType.DMA,
            pltpu.SemaphoreType.DMA,
        ],
    )
    def body(a_hbm, n_hbm, lab_hbm, out_hbm, lab_v, a_v, n_v,
             acc_v, eidx_v, shared, sem_a, sem_n, sem_a1, sem_n1):
        cid = lax.axis_index("c")
        sid = lax.axis_index("s")
        pltpu.sync_copy(lab_hbm, lab_v.at[pl.ds(0, 2)])
        @pl.when(sid == 0)
        def _():
            acc_v[...] = jnp.zeros((16,), jnp.float32)
            pltpu.sync_copy(acc_v, shared)
        lv = lab_v[...]
        e = (lv[0] != lv[1]).astype(jnp.int32)
        row = ((3 + cid) * e) * _NS + sid
        half = _CHUNK // 2
        cp_a0 = pltpu.async_copy(
            a_hbm.at[row, pl.ds(0, half)], a_v.at[pl.ds(0, half)], sem_a)
        cp_n0 = pltpu.async_copy(
            n_hbm.at[row, pl.ds(0, half)], n_v.at[pl.ds(0, half)], sem_n)
        cp_a1 = pltpu.async_copy(
            a_hbm.at[row, pl.ds(half, half)], a_v.at[pl.ds(half, half)],
            sem_a1)
        cp_n1 = pltpu.async_copy(
            n_hbm.at[row, pl.ds(half, half)], n_v.at[pl.ds(half, half)],
            sem_n1)

        def step(i, accs):
            base = pl.multiple_of(i * (16 * _UNROLL), 16 * _UNROLL)
            new = []
            for j in range(_UNROLL):
                a = a_v[pl.ds(base + j * 16, 16)]
                n = n_v[pl.ds(base + j * 16, 16)]
                d = a - n + _EPS
                new.append(accs[j] + d * d)
            return tuple(new)

        zeros = jnp.zeros((16,), jnp.float32)
        nsteps = half // (16 * _UNROLL)
        cp_a0.wait()
        cp_n0.wait()
        accs = lax.fori_loop(0, nsteps, step, (zeros,) * _UNROLL)
        cp_a1.wait()
        cp_n1.wait()
        accs = lax.fori_loop(nsteps, 2 * nsteps, step, accs)
        acc = accs[0]
        for j in range(1, _UNROLL):
            acc = acc + accs[j]
        acc_v[...] = acc
        eidx_v[...] = lax.iota(jnp.int32, 16)
        plsc.subcore_barrier()
        pltpu.sync_copy(acc_v, shared.at[eidx_v], add=True)
        plsc.subcore_barrier()

        @pl.when(sid == 0)
        def _():
            pltpu.sync_copy(shared, acc_v)
            tot = acc_v[...]
            lane = lax.iota(jnp.int32, 16)
            for sh in (8, 4, 2, 1):
                tot = tot + tot.at[lane ^ sh].get(mode="promise_in_bounds")
            y16 = jnp.maximum(_GAMMA * (tot - _M * _M), 0.0)
            t = jnp.exp(-y16)
            z = t * (1.0 - t * (0.5 - t * (1.0 / 3.0 - t * 0.25)))
            for _ in range(3):
                z = z - 1.0 + (1.0 + t) * jnp.exp(-z)
            acc_v[...] = y16 + z
            pltpu.sync_copy(acc_v, out_hbm.at[cid])

    return body(anchor2d, negative2d, labels2)


def kernel(anchor, positive, negative, labels):
    del positive
    a2d = anchor.reshape(8 * _NS, _CHUNK)
    n2d = negative.reshape(8 * _NS, _CHUNK)
    terms = _sc_loss_terms(a2d, n2d, labels.astype(jnp.int32))
    return 0.5 * (terms[0, 0] + terms[1, 0])


# speedup vs baseline: 1.4942x; 1.3267x over previous
import functools

import jax
import jax.numpy as jnp
from jax import lax
from jax.experimental import pallas as pl
from jax.experimental.pallas import tpu as pltpu
from jax.experimental.pallas import tpu_sc as plsc

_M = 0.25
_GAMMA = 64.0
_EPS = 1e-6
_D = 131072
_NS = 16
_CHUNK = _D // _NS
_UNROLL = 8


def _sc_loss_terms(anchor4d, negative4d, labels2):
    mesh = plsc.VectorSubcoreMesh(core_axis_name="c", subcore_axis_name="s")

    @functools.partial(
        pl.kernel,
        out_type=jax.ShapeDtypeStruct((2, 16), jnp.float32),
        mesh=mesh,
        scratch_types=[
            pltpu.VMEM((16,), jnp.int32),
            pltpu.VMEM((_CHUNK,), jnp.float32),
            pltpu.VMEM((_CHUNK,), jnp.float32),
            pltpu.VMEM((16,), jnp.float32),
            pltpu.VMEM((16,), jnp.int32),
            pltpu.VMEM_SHARED((16,), jnp.float32),
            pltpu.SemaphoreType.DMA,
            pltpu.SemaphoreType.DMA,
            pltpu.SemaphoreType.DMA,
            pltpu.SemaphoreType.DMA,
        ],
    )
    def body(a_hbm, n_hbm, lab_hbm, out_hbm, lab_v, a_v, n_v,
             acc_v, eidx_v, shared, sem_a, sem_n, sem_a1, sem_n1):
        cid = lax.axis_index("c")
        sid = lax.axis_index("s")
        pltpu.sync_copy(lab_hbm, lab_v.at[pl.ds(0, 2)])
        @pl.when(sid == 0)
        def _():
            acc_v[...] = jnp.zeros((16,), jnp.float32)
            pltpu.sync_copy(acc_v, shared)
        lv = lab_v[...]
        e = (lv[0] != lv[1]).astype(jnp.int32)
        i = cid * e
        j = (1 - cid) * e
        off = sid * _CHUNK
        half = _CHUNK // 2
        cp_a0 = pltpu.async_copy(
            a_hbm.at[i, j, j, pl.ds(off, half)],
            a_v.at[pl.ds(0, half)], sem_a)
        cp_n0 = pltpu.async_copy(
            n_hbm.at[i, j, j, pl.ds(off, half)],
            n_v.at[pl.ds(0, half)], sem_n)
        cp_a1 = pltpu.async_copy(
            a_hbm.at[i, j, j, pl.ds(off + half, half)],
            a_v.at[pl.ds(half, half)], sem_a1)
        cp_n1 = pltpu.async_copy(
            n_hbm.at[i, j, j, pl.ds(off + half, half)],
            n_v.at[pl.ds(half, half)], sem_n1)

        def step(i, accs):
            base = pl.multiple_of(i * (16 * _UNROLL), 16 * _UNROLL)
            new = []
            for j in range(_UNROLL):
                a = a_v[pl.ds(base + j * 16, 16)]
                n = n_v[pl.ds(base + j * 16, 16)]
                d = a - n + _EPS
                new.append(accs[j] + d * d)
            return tuple(new)

        zeros = jnp.zeros((16,), jnp.float32)
        nsteps = half // (16 * _UNROLL)
        cp_a0.wait()
        cp_n0.wait()
        accs = lax.fori_loop(0, nsteps, step, (zeros,) * _UNROLL)
        cp_a1.wait()
        cp_n1.wait()
        accs = lax.fori_loop(nsteps, 2 * nsteps, step, accs)
        acc = accs[0]
        for j in range(1, _UNROLL):
            acc = acc + accs[j]
        acc_v[...] = acc
        eidx_v[...] = lax.iota(jnp.int32, 16)
        plsc.subcore_barrier()
        pltpu.sync_copy(acc_v, shared.at[eidx_v], add=True)
        plsc.subcore_barrier()

        @pl.when(sid == 0)
        def _():
            pltpu.sync_copy(shared, acc_v)
            tot = acc_v[...]
            lane = lax.iota(jnp.int32, 16)
            for sh in (8, 4, 2, 1):
                tot = tot + tot.at[lane ^ sh].get(mode="promise_in_bounds")
            y16 = jnp.maximum(_GAMMA * (tot - _M * _M), 0.0)
            t = jnp.exp(-y16)
            z = t * (1.0 - t * (0.5 - t * (1.0 / 3.0 - t * 0.25)))
            for _ in range(3):
                z = z - 1.0 + (1.0 + t) * jnp.exp(-z)
            acc_v[...] = y16 + z
            pltpu.sync_copy(acc_v, out_hbm.at[cid])

    return body(anchor4d, negative4d, labels2)


def kernel(anchor, positive, negative, labels):
    del positive
    terms = _sc_loss_terms(anchor, negative, labels.astype(jnp.int32))
    return 0.5 * (terms[0, 0] + terms[1, 0])
